# baseline (speedup 1.0000x reference)
import jax
import jax.numpy as jnp
from jax.experimental import pallas as pl


def _matmul_block(x_ref, e_ref, o_ref):
    x = x_ref[...].astype(jnp.float32)
    o_ref[...] = jnp.dot(x, e_ref[...], preferred_element_type=jnp.float32)


def kernel(inputs, embeddings):
    M, K = inputs.shape
    _, N = embeddings.shape
    BM = 2048
    out = pl.pallas_call(
        _matmul_block,
        grid=(M // BM,),
        in_specs=[
            pl.BlockSpec((BM, K), lambda i: (i, 0)),
            pl.BlockSpec((K, N), lambda i: (0, 0)),
        ],
        out_specs=pl.BlockSpec((BM, N), lambda i: (i, 0)),
        out_shape=jax.ShapeDtypeStruct((M, N), jnp.float32),
    )(inputs, embeddings)
    return out[1:]
